# baseline (speedup 1.0000x reference)
import functools

import jax
import jax.numpy as jnp
from jax import lax
from jax.experimental import pallas as pl
from jax.experimental.pallas import tpu as pltpu
from jax.experimental.pallas import tpu_sc as plsc

B, S, E, V = 4, 2048, 768, 100000
NC, NS, L = 2, 16, 16
NW = NC * NS
SCHUNK = S // NW
EV = E // L
CH = 32
NCHUNK = (B * SCHUNK) // CH
NBUF = 3


def _make_kernel():
    mesh = plsc.VectorSubcoreMesh(core_axis_name="c", subcore_axis_name="s")

    @functools.partial(
        pl.kernel,
        out_type=jax.ShapeDtypeStruct((B, S, E), jnp.float32),
        mesh=mesh,
        scratch_types=[
            pltpu.VMEM((B, SCHUNK), jnp.int32),
            pltpu.VMEM((SCHUNK, E), jnp.float32),
            [pltpu.VMEM((CH, E), jnp.float32) for _ in range(NBUF)],
            [pltpu.SemaphoreType.DMA for _ in range(NBUF)],
            [pltpu.SemaphoreType.DMA for _ in range(NBUF)],
            pltpu.SemaphoreType.DMA,
        ],
    )
    def k(ids_hbm, tab_hbm, pos_hbm, out_hbm, idx_v, pos_v, bufs, gsems, wsems,
          psem):
        wid = lax.axis_index("s") * NC + lax.axis_index("c")
        s0 = wid * SCHUNK

        pos_cp = pltpu.async_copy(pos_hbm.at[pl.ds(s0, SCHUNK)], pos_v, psem)
        for b in range(B):
            pltpu.sync_copy(ids_hbm.at[b, pl.ds(s0, SCHUNK)], idx_v.at[b])

        def gather(c):
            b, h = c // 2, c % 2
            return pltpu.async_copy(
                tab_hbm.at[idx_v.at[b, pl.ds(h * CH, CH)]],
                bufs[c % NBUF],
                gsems[c % NBUF],
            )

        HW = CH // 2

        def write_half(c, half):
            b, h = c // 2, c % 2
            return pltpu.async_copy(
                bufs[c % NBUF].at[pl.ds(half * HW, HW)],
                out_hbm.at[b, pl.ds(s0 + h * CH + half * HW, HW)],
                wsems[c % NBUF],
            )

        g_cp = [None] * NCHUNK
        w_cp = [None] * NCHUNK
        g_cp[0] = gather(0)
        g_cp[1] = gather(1)
        pos_cp.wait()

        for c in range(NCHUNK):
            g_cp[c].wait()

            h = c % 2
            buf = bufs[c % NBUF]

            def addseg(lo, hi):
                @plsc.parallel_loop(lo, hi, 1)
                def add_row(r):
                    pr = h * CH + r

                    @plsc.parallel_loop(0, E, 12 * L)
                    def add_cols(cb):
                        for kk in range(12):
                            plsc.addupdate(
                                buf.at[r, pl.ds(cb + kk * L, L)],
                                pos_v[pr, pl.ds(cb + kk * L, L)],
                            )

            addseg(0, CH // 4)
            nc = c + 2
            if nc < NCHUNK:
                if c >= 1:
                    for w in w_cp[c - 1]:
                        w.wait()
                g_cp[nc] = gather(nc)
            addseg(CH // 4, HW)
            w0 = write_half(c, 0)
            addseg(HW, CH)
            w1 = write_half(c, 1)
            w_cp[c] = (w0, w1)

        for c in (NCHUNK - 3, NCHUNK - 2, NCHUNK - 1):
            for w in w_cp[c]:
                w.wait()

    return k


_kernel = _make_kernel()


def kernel(input_ids, token_embeddings, position_embeddings):
    return _kernel(input_ids.astype(jnp.int32), token_embeddings,
                   position_embeddings)


# speedup vs baseline: 1.0624x; 1.0624x over previous
import functools

import jax
import jax.numpy as jnp
from jax import lax
from jax.experimental import pallas as pl
from jax.experimental.pallas import tpu as pltpu
from jax.experimental.pallas import tpu_sc as plsc

B, S, E, V = 4, 2048, 768, 100000
NC, NS, L = 2, 16, 16
NW = NC * NS
SCHUNK = S // NW
EV = E // L
CS = 8
NCHUNK = SCHUNK // CS
NBUF = 3


def _make_kernel():
    mesh = plsc.VectorSubcoreMesh(core_axis_name="c", subcore_axis_name="s")

    @functools.partial(
        pl.kernel,
        out_type=jax.ShapeDtypeStruct((B, S, E), jnp.float32),
        mesh=mesh,
        scratch_types=[
            pltpu.VMEM((B, SCHUNK), jnp.int32),
            pltpu.VMEM((SCHUNK, E), jnp.float32),
            [pltpu.VMEM((B, CS, E), jnp.float32) for _ in range(NBUF)],
            [pltpu.SemaphoreType.DMA for _ in range(NBUF)],
            [pltpu.SemaphoreType.DMA for _ in range(NBUF)],
            pltpu.SemaphoreType.DMA,
        ],
    )
    def k(ids_hbm, tab_hbm, pos_hbm, out_hbm, idx_v, pos_v, bufs, gsems, wsems,
          psem):
        wid = lax.axis_index("s") * NC + lax.axis_index("c")
        s0 = wid * SCHUNK

        pos_cp = pltpu.async_copy(pos_hbm.at[pl.ds(s0, SCHUNK)], pos_v, psem)
        for b in range(B):
            pltpu.sync_copy(ids_hbm.at[b, pl.ds(s0, SCHUNK)], idx_v.at[b])

        def gather(c):
            return [
                pltpu.async_copy(
                    tab_hbm.at[idx_v.at[b, pl.ds(c * CS, CS)]],
                    bufs[c % NBUF].at[b],
                    gsems[c % NBUF],
                )
                for b in range(B)
            ]

        def write(c):
            return [
                pltpu.async_copy(
                    bufs[c % NBUF].at[b],
                    out_hbm.at[b, pl.ds(s0 + c * CS, CS)],
                    wsems[c % NBUF],
                )
                for b in range(B)
            ]

        g_cp = [None] * NCHUNK
        w_cp = [None] * NCHUNK
        g_cp[0] = gather(0)
        g_cp[1] = gather(1)
        pos_cp.wait()

        for c in range(NCHUNK):
            for g in g_cp[c]:
                g.wait()

            buf = bufs[c % NBUF]

            @plsc.parallel_loop(0, CS, 1)
            def add_row(sl):
                pr = c * CS + sl
                for e in range(EV):
                    pv = pos_v[pr, pl.ds(e * L, L)]
                    for b in range(B):
                        buf[b, sl, pl.ds(e * L, L)] = (
                            buf[b, sl, pl.ds(e * L, L)] + pv
                        )

            nc = c + 2
            if nc < NCHUNK:
                if c >= 1:
                    for w in w_cp[c - 1]:
                        w.wait()
                g_cp[nc] = gather(nc)

            w_cp[c] = write(c)

        for c in (NCHUNK - 3, NCHUNK - 2, NCHUNK - 1):
            for w in w_cp[c]:
                w.wait()

    return k


_kernel = _make_kernel()


def kernel(input_ids, token_embeddings, position_embeddings):
    return _kernel(input_ids.astype(jnp.int32), token_embeddings,
                   position_embeddings)
